# baseline (speedup 1.0000x reference)
import jax
import jax.numpy as jnp
from jax.experimental import pallas as pl
from jax.experimental.pallas import tpu as pltpu

_B = 128
_FEAT = 3 * 384 * 384
_CHUNK = 6912
_NSTEP = _FEAT // _CHUNK
_REM = 115


def _mse_body(xr_ref, x_ref, out_ref):
    j = pl.program_id(0)

    @pl.when(j == 0)
    def _init():
        out_ref[...] = jnp.zeros_like(out_ref)

    xv = x_ref[...]
    d0 = xr_ref[0] - xv
    d1 = xr_ref[1] - xv
    p0 = jnp.sum(d0 * d0, axis=1, keepdims=True)
    p1 = jnp.sum(d1 * d1, axis=1, keepdims=True)
    out_ref[:, 0:1] += p0
    out_ref[:, 1:2] += p1


def _select_body(lr_ref, lc_ref, out_ref):
    inv = 1.0 / jnp.float32(_FEAT)
    l0r = lr_ref[0:1, :] * inv
    l0c = lc_ref[:, 0:1] * inv
    l1c = lc_ref[:, 1:2] * inv
    rowv = jnp.broadcast_to(l0c, (_B, _B))
    colv = jnp.broadcast_to(l0r, (_B, _B))
    rr = jax.lax.broadcasted_iota(jnp.int32, (_B, _B), 0)
    cc = jax.lax.broadcasted_iota(jnp.int32, (_B, _B), 1)
    smaller = (colv < rowv) | ((colv == rowv) & (cc < rr))
    rank = jnp.sum(smaller.astype(jnp.float32), axis=1, keepdims=True)
    sel = (rank < jnp.float32(_REM)).astype(jnp.float32)
    total = jnp.sum(sel * (l0c + l1c), keepdims=True)
    out_ref[...] = total.reshape(1, 1) / jnp.float32(_REM)


def kernel(xr, x):
    xr3 = xr.reshape(2, _B, _FEAT)
    x2 = x.reshape(_B, _FEAT)
    acc = pl.pallas_call(
        _mse_body,
        grid=(_NSTEP,),
        in_specs=[
            pl.BlockSpec((2, _B, _CHUNK), lambda j: (0, 0, j)),
            pl.BlockSpec((_B, _CHUNK), lambda j: (0, j)),
        ],
        out_specs=pl.BlockSpec((_B, 2), lambda j: (0, 0)),
        out_shape=jax.ShapeDtypeStruct((_B, 2), jnp.float32),
        compiler_params=pltpu.CompilerParams(
            dimension_semantics=("arbitrary",),
        ),
    )(xr3, x2)

    loss = pl.pallas_call(
        _select_body,
        in_specs=[
            pl.BlockSpec((2, _B), lambda: (0, 0)),
            pl.BlockSpec((_B, 2), lambda: (0, 0)),
        ],
        out_specs=pl.BlockSpec((1, 1), lambda: (0, 0)),
        out_shape=jax.ShapeDtypeStruct((1, 1), jnp.float32),
    )(jnp.transpose(acc), acc)
    return loss[0, 0]
